# baseline (speedup 1.0000x reference)
import functools

import jax
import jax.numpy as jnp
from jax import lax
from jax.experimental import pallas as pl
from jax.experimental.pallas import tpu as pltpu
from jax.experimental.pallas import tpu_sc as plsc


def kernel(x, table):
    B, S = x.shape
    V, D = table.shape
    RPS = 128 // D

    NW = 32
    b_per_w = B // NW
    NG = b_per_w // 128

    xt = jnp.swapaxes(x, 0, 1).astype(jnp.int32)
    table2 = table.reshape(V // RPS, D * RPS)

    mesh = plsc.VectorSubcoreMesh(core_axis_name="c", subcore_axis_name="s")

    @functools.partial(
        pl.kernel,
        mesh=mesh,
        out_type=jax.ShapeDtypeStruct((S, D, B), jnp.float32),
        compiler_params=pltpu.CompilerParams(
            use_tc_tiling_on_sc=True, needs_layout_passes=False),
        scratch_types=[
            pltpu.VMEM((S, b_per_w), jnp.int32),
            pltpu.VMEM((NG, 128), jnp.int32),
            pltpu.VMEM((b_per_w, 128), jnp.float32),
            pltpu.VMEM((2, D, b_per_w), jnp.float32),
            pltpu.SemaphoreType.DMA,
            pltpu.SemaphoreType.DMA,
        ],
    )
    def emb(idx_hbm, table_hbm, out_hbm, idx_v, list_v, g_v, sel_v, gsem, osem):
        wid = lax.axis_index("s") * 2 + lax.axis_index("c")
        wb = wid * b_per_w
        pltpu.sync_copy(idx_hbm.at[:, pl.ds(wb, b_per_w)], idx_v)
        lane = lax.iota(jnp.int32, 16)

        def body(s, carry):
            p = lax.rem(s, 2)

            for k in range(b_per_w // 16):
                v = idx_v[s, pl.ds(k * 16, 16)]
                list_v[k // 8, pl.ds((k % 8) * 16, 16)] = (
                    lax.shift_right_logical(v, 2))

            copies = [
                pltpu.async_copy(
                    table_hbm.at[list_v.at[c]],
                    g_v.at[pl.ds(c * 128, 128)], gsem)
                for c in range(NG)
            ]
            for c in copies:
                c.wait()

            @pl.when(s >= 2)
            def _wait_prev_out():
                pltpu.make_async_copy(
                    sel_v.at[p], out_hbm.at[0, :, pl.ds(wb, b_per_w)], osem
                ).wait()

            @plsc.parallel_loop(0, b_per_w // 16, 1, unroll=4)
            def select(jg):
                j_vec = jg * 16 + lane
                o_vec = (idx_v[s, pl.ds(jg * 16, 16)] & 3) * D
                for d in range(D):
                    vals = plsc.load_gather(g_v, [j_vec, o_vec + d])
                    sel_v[p, d, pl.ds(jg * 16, 16)] = vals

            pltpu.async_copy(
                sel_v.at[p], out_hbm.at[s, :, pl.ds(wb, b_per_w)], osem)
            return carry

        lax.fori_loop(0, S, body, 0)
        pltpu.make_async_copy(
            sel_v.at[0], out_hbm.at[0, :, pl.ds(wb, b_per_w)], osem).wait()
        pltpu.make_async_copy(
            sel_v.at[1], out_hbm.at[0, :, pl.ds(wb, b_per_w)], osem).wait()

    out = emb(xt, table2)
    return jnp.transpose(out, (2, 0, 1))


# speedup vs baseline: 1.1168x; 1.1168x over previous
import functools

import jax
import jax.numpy as jnp
from jax import lax
from jax.experimental import pallas as pl
from jax.experimental.pallas import tpu as pltpu
from jax.experimental.pallas import tpu_sc as plsc


def kernel(x, table):
    B, S = x.shape
    V, D = table.shape

    NW = 32
    b_per_w = B // NW

    xt = jnp.swapaxes(x, 0, 1).astype(jnp.int32)

    mesh = plsc.VectorSubcoreMesh(core_axis_name="c", subcore_axis_name="s")

    @functools.partial(
        pl.kernel,
        mesh=mesh,
        out_type=jax.ShapeDtypeStruct((S, B, D), jnp.float32),
        compiler_params=pltpu.CompilerParams(use_tc_tiling_on_sc=False),
        scratch_types=[
            pltpu.VMEM((S, b_per_w), jnp.int32),
            pltpu.VMEM((2, b_per_w, D), jnp.float32),
            pltpu.SemaphoreType.DMA,
            pltpu.SemaphoreType.DMA,
        ],
    )
    def emb(idx_hbm, table_hbm, out_hbm, idx_v, rows_v, gsem, osem):
        wid = lax.axis_index("s") * 2 + lax.axis_index("c")
        wb = wid * b_per_w
        pltpu.sync_copy(idx_hbm.at[:, pl.ds(wb, b_per_w)], idx_v)

        def body(s, carry):
            p = lax.rem(s, 2)

            @pl.when(s >= 2)
            def _wait_prev_out():
                pltpu.make_async_copy(
                    rows_v.at[p], out_hbm.at[0, pl.ds(wb, b_per_w)], osem
                ).wait()

            pltpu.async_copy(
                table_hbm.at[idx_v.at[s]], rows_v.at[p], gsem).wait()
            pltpu.async_copy(
                rows_v.at[p], out_hbm.at[s, pl.ds(wb, b_per_w)], osem)
            return carry

        lax.fori_loop(0, S, body, 0)
        pltpu.make_async_copy(
            rows_v.at[0], out_hbm.at[0, pl.ds(wb, b_per_w)], osem).wait()
        pltpu.make_async_copy(
            rows_v.at[1], out_hbm.at[0, pl.ds(wb, b_per_w)], osem).wait()

    out = emb(xt, table)
    return jnp.swapaxes(out, 0, 1)
